# baseline (speedup 1.0000x reference)
import functools

import jax
import jax.numpy as jnp
from jax import lax
from jax.experimental import pallas as pl
from jax.experimental.pallas import tpu as pltpu
from jax.experimental.pallas import tpu_sc as plsc

N = 10000
E = 320000
D = 128
H = 128
C = 10
B = 64
BOND = 10

NC = 2
NS = 16
NW = NC * NS

EPW = E // NW
SUB = 80
WIN = 2000
WSUB = WIN // SUB
NSTG = EPW // WIN
RPS = 624
REM = N - NS * RPS

_f32 = jnp.float32
_i32 = jnp.int32


def _sc_body(vals_hbm, src_hbm, dst_hbm, ew_hbm, zeros_hbm, out_hbm,
             src_v, dst_v, ew_v, rows_v, scaled_v, agg_sh,
             gsem0, gsem1, ssem0, ssem1):
    c = lax.axis_index("c")
    s = lax.axis_index("s")
    wid = s * NC + c
    gsems = (gsem0, gsem1)
    ssems = (ssem0, ssem1)

    pltpu.sync_copy(zeros_hbm, agg_sh.at[pl.ds(s * RPS, RPS)])

    @pl.when(s == NS - 1)
    def _():
        pltpu.sync_copy(zeros_hbm.at[pl.ds(0, REM)],
                        agg_sh.at[pl.ds(NS * RPS, REM)])

    plsc.subcore_barrier()

    ebase = wid * EPW

    def g_copy(r, b):
        off = pl.multiple_of(r * SUB, SUB)
        return pltpu.make_async_copy(vals_hbm.at[src_v.at[pl.ds(off, SUB)]],
                                     rows_v.at[b], gsems[b])

    def s_copy(r, b):
        return pltpu.make_async_copy(scaled_v.at[b],
                                     agg_sh.at[dst_v.at[r, 0]], ssems[b])

    def scale(r, b):
        rbase = pl.multiple_of(r * SUB, SUB)

        @plsc.parallel_loop(0, SUB, step=1, unroll=8)
        def _(e):
            egrp = (e // 16) * 16
            ew16 = ew_v[pl.ds(pl.multiple_of(rbase + egrp, 16), 16)]
            lane = jnp.full((16,), e - egrp, _i32)
            splat = lax.gather(
                ew16, lane[:, None],
                lax.GatherDimensionNumbers(offset_dims=(),
                                           collapsed_slice_dims=(0,),
                                           start_index_map=(0,)),
                slice_sizes=(1,),
                mode=lax.GatherScatterMode.PROMISE_IN_BOUNDS)
            for cb in range(D // 32):
                w = plsc.bitcast(rows_v.at[b][e, pl.ds(32 * cb, 32)], _i32)
                va = plsc.bitcast(w << 16, _f32)
                vb = plsc.bitcast(w & jnp.int32(-65536), _f32)
                scaled_v.at[b][e, pl.ds(32 * cb, 16)] = va * splat
                scaled_v.at[b][e, pl.ds(32 * cb + 16, 16)] = vb * splat

    def s_start(r, b):
        pltpu.async_copy(scaled_v.at[b], agg_sh.at[dst_v.at[r, 0]], ssems[b],
                         add=True)

    for stage in range(NSTG):
        w0 = ebase + stage * WIN
        pltpu.sync_copy(src_hbm.at[pl.ds(w0, WIN)], src_v)
        pltpu.sync_copy(ew_hbm.at[pl.ds(w0, WIN)], ew_v)
        pltpu.sync_copy(dst_hbm.at[pl.ds(w0 // SUB, WSUB)], dst_v)

        b0 = stage % 2
        b1 = (stage + 1) % 2
        g_copy(0, b0).start()
        g_copy(1, b1).start()

        def pair_body(i, carry, _b0=b0, _b1=b1):
            for b, bb in ((0, _b0), (1, _b1)):
                r = 2 * i + b
                g_copy(r, bb).wait()

                @pl.when(r >= 2)
                def _():
                    s_copy(r - 2, bb).wait()

                scale(r, bb)
                s_start(r, bb)

                @pl.when(r + 2 <= WSUB - 1)
                def _():
                    g_copy(r + 2, bb).start()
            return carry

        lax.fori_loop(0, (WSUB - 1) // 2, pair_body, 0)

        rl = WSUB - 1
        g_copy(rl, b0).wait()
        s_copy(rl - 2, b0).wait()
        scale(rl, b0)
        s_start(rl, b0)
        s_copy(rl - 1, b1).wait()
        s_copy(rl, b0).wait()

    plsc.subcore_barrier()
    pltpu.sync_copy(agg_sh.at[pl.ds(s * RPS, RPS)],
                    out_hbm.at[c].at[pl.ds(s * RPS, RPS)])

    @pl.when(s == NS - 1)
    def _():
        pltpu.sync_copy(agg_sh.at[pl.ds(NS * RPS, REM)],
                        out_hbm.at[c].at[pl.ds(NS * RPS, REM)])


def _sc_segment(vals, src, dst3, ew, zeros):
    mesh = plsc.VectorSubcoreMesh(core_axis_name="c", subcore_axis_name="s",
                                  num_cores=NC, num_subcores=NS)
    fn = pl.kernel(
        _sc_body,
        out_type=jax.ShapeDtypeStruct((NC, N, D), _f32),
        mesh=mesh,
        scratch_types=[
            pltpu.VMEM((WIN,), _i32),
            pltpu.VMEM((WSUB, 1, SUB), _i32),
            pltpu.VMEM((WIN,), _f32),
            pltpu.VMEM((2, SUB, D), jnp.bfloat16),
            pltpu.VMEM((2, SUB, D), _f32),
            pltpu.VMEM_SHARED((N, D), _f32),
            pltpu.SemaphoreType.DMA,
            pltpu.SemaphoreType.DMA,
            pltpu.SemaphoreType.DMA,
            pltpu.SemaphoreType.DMA,
        ],
        compiler_params=pltpu.CompilerParams(needs_layout_passes=False,
                                             use_tc_tiling_on_sc=False),
    )
    return fn(vals, src, dst3, ew, zeros)


EW_ROWS = 640
EW_COLS = 500
EW_BLK_R = 80


def _ew_body(ea_ref, w_ref, b_ref, out_ref):
    acc = ea_ref[0] * w_ref[0:1, 0:1]
    for k in range(1, BOND):
        acc += ea_ref[k] * w_ref[0:1, k:k + 1]
    out_ref[...] = acc + b_ref[0:1, 0:1]


def _ew_call(eaT3, w_row, b11):
    grid = EW_ROWS // EW_BLK_R
    return pl.pallas_call(
        _ew_body,
        grid=(grid,),
        in_specs=[
            pl.BlockSpec((BOND, EW_BLK_R, EW_COLS), lambda i: (0, i, 0)),
            pl.BlockSpec((1, BOND), lambda i: (0, 0)),
            pl.BlockSpec((1, 1), lambda i: (0, 0)),
        ],
        out_specs=pl.BlockSpec((EW_BLK_R, EW_COLS), lambda i: (i, 0)),
        out_shape=jax.ShapeDtypeStruct((EW_ROWS, EW_COLS), _f32),
    )(eaT3, w_row, b11)


RB = 1000


def _dense_body(aggp_ref, x_ref, batch_ref, wrel_ref, brel_ref, wroot_ref,
                r_ref, sum_ref, max_ref, cnt_ref):
    i = pl.program_id(0)
    agg = aggp_ref[0] + aggp_ref[1]
    h = (lax.dot_general(agg, wrel_ref[...], (((1,), (0,)), ((), ())),
                         preferred_element_type=_f32)
         + brel_ref[...]
         + lax.dot_general(x_ref[...], wroot_ref[...], (((1,), (0,)), ((), ())),
                           preferred_element_type=_f32))
    r_ref[...] = jnp.maximum(h, 0.0)

    onehot = (batch_ref[...] ==
              lax.broadcasted_iota(_i32, (1, B), 1)).astype(_f32)
    sums = lax.dot_general(onehot, h, (((0,), (0,)), ((), ())),
                           preferred_element_type=_f32)
    cnts = lax.dot_general(onehot, jnp.ones((RB, D), _f32),
                           (((0,), (0,)), ((), ())),
                           preferred_element_type=_f32)

    @pl.when(i == 0)
    def _():
        sum_ref[...] = jnp.zeros_like(sum_ref)
        cnt_ref[...] = jnp.zeros_like(cnt_ref)
        max_ref[...] = jnp.full_like(max_ref, -jnp.inf)

    sum_ref[...] += sums
    cnt_ref[...] += cnts

    lo = batch_ref[0, 0]
    hi = batch_ref[RB - 1, 0]
    giota = lax.broadcasted_iota(_i32, (B, 1), 0)

    def gbody(g, carry):
        m = batch_ref[...] == g
        mg = jnp.max(jnp.where(m, h, -jnp.inf), axis=0, keepdims=True)
        max_ref[...] = jnp.where(giota == g,
                                 jnp.maximum(max_ref[...], mg), max_ref[...])
        return carry

    lax.fori_loop(lo, hi + 1, gbody, 0)


def _dense_call(aggp, x, batch_col, wrel, brel_row, wroot):
    grid = N // RB
    return pl.pallas_call(
        _dense_body,
        grid=(grid,),
        in_specs=[
            pl.BlockSpec((NC, RB, D), lambda i: (0, i, 0)),
            pl.BlockSpec((RB, D), lambda i: (i, 0)),
            pl.BlockSpec((RB, 1), lambda i: (i, 0)),
            pl.BlockSpec((D, H), lambda i: (0, 0)),
            pl.BlockSpec((1, H), lambda i: (0, 0)),
            pl.BlockSpec((D, H), lambda i: (0, 0)),
        ],
        out_specs=[
            pl.BlockSpec((RB, H), lambda i: (i, 0)),
            pl.BlockSpec((B, H), lambda i: (0, 0)),
            pl.BlockSpec((B, H), lambda i: (0, 0)),
            pl.BlockSpec((B, H), lambda i: (0, 0)),
        ],
        out_shape=[
            jax.ShapeDtypeStruct((N, H), _f32),
            jax.ShapeDtypeStruct((B, H), _f32),
            jax.ShapeDtypeStruct((B, H), _f32),
            jax.ShapeDtypeStruct((B, H), _f32),
        ],
        compiler_params=pltpu.CompilerParams(
            dimension_semantics=("arbitrary",)),
    )(aggp, x, batch_col, wrel, brel_row, wroot)


def _mlp_body(mx1_ref, sum1_ref, mx2_ref, sum2_ref, cnt_ref,
              wl1_ref, bl1_ref, wl2_ref, bl2_ref, out_ref):
    z_mx = mx1_ref[...] + mx2_ref[...]
    z_mean = (sum1_ref[...] + sum2_ref[...]) / jnp.maximum(cnt_ref[...], 1.0)
    z = (lax.dot_general(z_mx, wl1_ref[0:H], (((1,), (0,)), ((), ())),
                         preferred_element_type=_f32)
         + lax.dot_general(z_mean, wl1_ref[H:2 * H], (((1,), (0,)), ((), ())),
                           preferred_element_type=_f32)
         + bl1_ref[...])
    z = jnp.maximum(z, 0.0)
    out_ref[...] = lax.dot_general(z, wl2_ref[...], (((1,), (0,)), ((), ())),
                                   preferred_element_type=_f32) + bl2_ref[...]


def _mlp_call(mx1, sum1, mx2, sum2, cnt, wl1, bl1_row, wl2, bl2_row):
    return pl.pallas_call(
        _mlp_body,
        out_shape=jax.ShapeDtypeStruct((B, C), _f32),
    )(mx1, sum1, mx2, sum2, cnt, wl1, bl1_row, wl2, bl2_row)


def kernel(x, edge_index, edge_a, batch, W_be, b_be, W1_rel, b1_rel, W1_root,
           W2_rel, b2_rel, W2_root, Wl1, bl1, Wl2, bl2):
    src = edge_index[0]
    dst3 = edge_index[1].reshape(E // SUB, 1, SUB)

    eaT3 = edge_a.T.reshape(BOND, EW_ROWS, EW_COLS)
    ew = _ew_call(eaT3, W_be.reshape(1, BOND), b_be.reshape(1, 1)).reshape(E)

    zeros = jnp.zeros((RPS, D), _f32)
    batch_col = batch.reshape(N, 1)

    def _half_interleave(v):
        return (v.reshape(N, D // 32, 2, 16).swapaxes(2, 3)
                .reshape(N, D).astype(jnp.bfloat16))

    aggp1 = _sc_segment(_half_interleave(x), src, dst3, ew, zeros)
    r1, sum1, max1, cnt = _dense_call(aggp1, x, batch_col, W1_rel,
                                      b1_rel.reshape(1, H), W1_root)

    aggp2 = _sc_segment(_half_interleave(r1), src, dst3, ew, zeros)
    _, sum2, max2, _ = _dense_call(aggp2, r1, batch_col, W2_rel,
                                   b2_rel.reshape(1, H), W2_root)

    return _mlp_call(max1, sum1, max2, sum2, cnt, Wl1, bl1.reshape(1, H),
                     Wl2, bl2.reshape(1, C))


# speedup vs baseline: 1.0414x; 1.0414x over previous
import functools

import jax
import jax.numpy as jnp
from jax import lax
from jax.experimental import pallas as pl
from jax.experimental.pallas import tpu as pltpu
from jax.experimental.pallas import tpu_sc as plsc

N = 10000
E = 320000
D = 128
H = 128
C = 10
B = 64
BOND = 10

NC = 2
NS = 16
NW = NC * NS

EPW = E // NW
SUB = 80
WIN = 2000
WSUB = WIN // SUB
NSTG = EPW // WIN
RPS = 624
REM = N - NS * RPS

_f32 = jnp.float32
_i32 = jnp.int32


def _sc_body(vals_hbm, src_hbm, dst_hbm, ew_hbm, zeros_hbm, out_hbm,
             src_v, dst_v, ew_v, rows_v, scaled_v, agg_sh,
             gsem0, gsem1, ssem0, ssem1):
    c = lax.axis_index("c")
    s = lax.axis_index("s")
    wid = s * NC + c
    gsems = (gsem0, gsem1)
    ssems = (ssem0, ssem1)

    pltpu.sync_copy(zeros_hbm, agg_sh.at[pl.ds(s * RPS, RPS)])

    @pl.when(s == NS - 1)
    def _():
        pltpu.sync_copy(zeros_hbm.at[pl.ds(0, REM)],
                        agg_sh.at[pl.ds(NS * RPS, REM)])

    plsc.subcore_barrier()

    ebase = wid * EPW

    def g_copy(r, b):
        off = pl.multiple_of(r * SUB, SUB)
        return pltpu.make_async_copy(vals_hbm.at[src_v.at[pl.ds(off, SUB)]],
                                     rows_v.at[b], gsems[b])

    def s_copy(r, b):
        return pltpu.make_async_copy(scaled_v.at[b],
                                     agg_sh.at[dst_v.at[r, 0]], ssems[b])

    def scale(r, b):
        rbase = pl.multiple_of(r * SUB, SUB)

        @plsc.parallel_loop(0, SUB, step=1, unroll=8)
        def _(e):
            egrp = (e // 16) * 16
            ew16 = ew_v[pl.ds(pl.multiple_of(rbase + egrp, 16), 16)]
            lane = jnp.full((16,), e - egrp, _i32)
            splat = lax.gather(
                ew16, lane[:, None],
                lax.GatherDimensionNumbers(offset_dims=(),
                                           collapsed_slice_dims=(0,),
                                           start_index_map=(0,)),
                slice_sizes=(1,),
                mode=lax.GatherScatterMode.PROMISE_IN_BOUNDS)
            for cc in range(D // 16):
                sl = pl.ds(16 * cc, 16)
                scaled_v.at[b][e, sl] = rows_v.at[b][e, sl] * splat

    def s_start(r, b):
        pltpu.async_copy(scaled_v.at[b], agg_sh.at[dst_v.at[r, 0]], ssems[b],
                         add=True)

    for stage in range(NSTG):
        w0 = ebase + stage * WIN
        pltpu.sync_copy(src_hbm.at[pl.ds(w0, WIN)], src_v)
        pltpu.sync_copy(ew_hbm.at[pl.ds(w0, WIN)], ew_v)
        pltpu.sync_copy(dst_hbm.at[pl.ds(w0 // SUB, WSUB)], dst_v)

        b0 = stage % 2
        b1 = (stage + 1) % 2
        g_copy(0, b0).start()
        g_copy(1, b1).start()

        def pair_body(i, carry, _b0=b0, _b1=b1):
            for b, bb in ((0, _b0), (1, _b1)):
                r = 2 * i + b
                g_copy(r, bb).wait()

                @pl.when(r >= 2)
                def _():
                    s_copy(r - 2, bb).wait()

                scale(r, bb)
                s_start(r, bb)

                @pl.when(r + 2 <= WSUB - 1)
                def _():
                    g_copy(r + 2, bb).start()
            return carry

        lax.fori_loop(0, (WSUB - 1) // 2, pair_body, 0)

        rl = WSUB - 1
        g_copy(rl, b0).wait()
        s_copy(rl - 2, b0).wait()
        scale(rl, b0)
        s_start(rl, b0)
        s_copy(rl - 1, b1).wait()
        s_copy(rl, b0).wait()

    plsc.subcore_barrier()
    pltpu.sync_copy(agg_sh.at[pl.ds(s * RPS, RPS)],
                    out_hbm.at[c].at[pl.ds(s * RPS, RPS)])

    @pl.when(s == NS - 1)
    def _():
        pltpu.sync_copy(agg_sh.at[pl.ds(NS * RPS, REM)],
                        out_hbm.at[c].at[pl.ds(NS * RPS, REM)])


def _sc_segment(vals, src, dst3, ew, zeros):
    mesh = plsc.VectorSubcoreMesh(core_axis_name="c", subcore_axis_name="s",
                                  num_cores=NC, num_subcores=NS)
    fn = pl.kernel(
        _sc_body,
        out_type=jax.ShapeDtypeStruct((NC, N, D), _f32),
        mesh=mesh,
        scratch_types=[
            pltpu.VMEM((WIN,), _i32),
            pltpu.VMEM((WSUB, 1, SUB), _i32),
            pltpu.VMEM((WIN,), _f32),
            pltpu.VMEM((2, SUB, D), _f32),
            pltpu.VMEM((2, SUB, D), _f32),
            pltpu.VMEM_SHARED((N, D), _f32),
            pltpu.SemaphoreType.DMA,
            pltpu.SemaphoreType.DMA,
            pltpu.SemaphoreType.DMA,
            pltpu.SemaphoreType.DMA,
        ],
        compiler_params=pltpu.CompilerParams(needs_layout_passes=False),
    )
    return fn(vals, src, dst3, ew, zeros)


EW_ROWS = 640
EW_COLS = 500
EW_BLK_R = 80


def _ew_body(ea_ref, w_ref, b_ref, out_ref):
    acc = ea_ref[0] * w_ref[0:1, 0:1]
    for k in range(1, BOND):
        acc += ea_ref[k] * w_ref[0:1, k:k + 1]
    out_ref[...] = acc + b_ref[0:1, 0:1]


def _ew_call(eaT3, w_row, b11):
    grid = EW_ROWS // EW_BLK_R
    return pl.pallas_call(
        _ew_body,
        grid=(grid,),
        in_specs=[
            pl.BlockSpec((BOND, EW_BLK_R, EW_COLS), lambda i: (0, i, 0)),
            pl.BlockSpec((1, BOND), lambda i: (0, 0)),
            pl.BlockSpec((1, 1), lambda i: (0, 0)),
        ],
        out_specs=pl.BlockSpec((EW_BLK_R, EW_COLS), lambda i: (i, 0)),
        out_shape=jax.ShapeDtypeStruct((EW_ROWS, EW_COLS), _f32),
    )(eaT3, w_row, b11)


RB = 1000


def _dense_body(aggp_ref, x_ref, batch_ref, wrel_ref, brel_ref, wroot_ref,
                r_ref, sum_ref, max_ref, cnt_ref):
    i = pl.program_id(0)
    agg = aggp_ref[0] + aggp_ref[1]
    h = (lax.dot_general(agg, wrel_ref[...], (((1,), (0,)), ((), ())),
                         preferred_element_type=_f32)
         + brel_ref[...]
         + lax.dot_general(x_ref[...], wroot_ref[...], (((1,), (0,)), ((), ())),
                           preferred_element_type=_f32))
    r_ref[...] = jnp.maximum(h, 0.0)

    onehot = (batch_ref[...] ==
              lax.broadcasted_iota(_i32, (1, B), 1)).astype(_f32)
    sums = lax.dot_general(onehot, h, (((0,), (0,)), ((), ())),
                           preferred_element_type=_f32)
    cnts = lax.dot_general(onehot, jnp.ones((RB, D), _f32),
                           (((0,), (0,)), ((), ())),
                           preferred_element_type=_f32)

    @pl.when(i == 0)
    def _():
        sum_ref[...] = jnp.zeros_like(sum_ref)
        cnt_ref[...] = jnp.zeros_like(cnt_ref)
        max_ref[...] = jnp.full_like(max_ref, -jnp.inf)

    sum_ref[...] += sums
    cnt_ref[...] += cnts

    lo = batch_ref[0, 0]
    hi = batch_ref[RB - 1, 0]
    giota = lax.broadcasted_iota(_i32, (B, 1), 0)

    def gbody(g, carry):
        m = batch_ref[...] == g
        mg = jnp.max(jnp.where(m, h, -jnp.inf), axis=0, keepdims=True)
        max_ref[...] = jnp.where(giota == g,
                                 jnp.maximum(max_ref[...], mg), max_ref[...])
        return carry

    lax.fori_loop(lo, hi + 1, gbody, 0)


def _dense_call(aggp, x, batch_col, wrel, brel_row, wroot):
    grid = N // RB
    return pl.pallas_call(
        _dense_body,
        grid=(grid,),
        in_specs=[
            pl.BlockSpec((NC, RB, D), lambda i: (0, i, 0)),
            pl.BlockSpec((RB, D), lambda i: (i, 0)),
            pl.BlockSpec((RB, 1), lambda i: (i, 0)),
            pl.BlockSpec((D, H), lambda i: (0, 0)),
            pl.BlockSpec((1, H), lambda i: (0, 0)),
            pl.BlockSpec((D, H), lambda i: (0, 0)),
        ],
        out_specs=[
            pl.BlockSpec((RB, H), lambda i: (i, 0)),
            pl.BlockSpec((B, H), lambda i: (0, 0)),
            pl.BlockSpec((B, H), lambda i: (0, 0)),
            pl.BlockSpec((B, H), lambda i: (0, 0)),
        ],
        out_shape=[
            jax.ShapeDtypeStruct((N, H), _f32),
            jax.ShapeDtypeStruct((B, H), _f32),
            jax.ShapeDtypeStruct((B, H), _f32),
            jax.ShapeDtypeStruct((B, H), _f32),
        ],
        compiler_params=pltpu.CompilerParams(
            dimension_semantics=("arbitrary",)),
    )(aggp, x, batch_col, wrel, brel_row, wroot)


def _dense2_body(aggp_ref, x_ref, batch_ref, wrel_ref, brel_ref, wroot_ref,
                 mx1_ref, sum1_ref, cnt_ref, wl1_ref, bl1_ref, wl2_ref,
                 bl2_ref, out_ref, sum_s, max_s):
    i = pl.program_id(0)
    agg = aggp_ref[0] + aggp_ref[1]
    h = (lax.dot_general(agg, wrel_ref[...], (((1,), (0,)), ((), ())),
                         preferred_element_type=_f32)
         + brel_ref[...]
         + lax.dot_general(x_ref[...], wroot_ref[...], (((1,), (0,)), ((), ())),
                           preferred_element_type=_f32))

    onehot = (batch_ref[...] ==
              lax.broadcasted_iota(_i32, (1, B), 1)).astype(_f32)
    sums = lax.dot_general(onehot, h, (((0,), (0,)), ((), ())),
                           preferred_element_type=_f32)

    @pl.when(i == 0)
    def _():
        sum_s[...] = jnp.zeros_like(sum_s)
        max_s[...] = jnp.full_like(max_s, -jnp.inf)

    sum_s[...] += sums

    lo = batch_ref[0, 0]
    hi = batch_ref[RB - 1, 0]
    giota = lax.broadcasted_iota(_i32, (B, 1), 0)

    def gbody(g, carry):
        m = batch_ref[...] == g
        mg = jnp.max(jnp.where(m, h, -jnp.inf), axis=0, keepdims=True)
        max_s[...] = jnp.where(giota == g,
                               jnp.maximum(max_s[...], mg), max_s[...])
        return carry

    lax.fori_loop(lo, hi + 1, gbody, 0)

    @pl.when(i == N // RB - 1)
    def _():
        z_mx = mx1_ref[...] + max_s[...]
        z_mean = ((sum1_ref[...] + sum_s[...])
                  / jnp.maximum(cnt_ref[...], 1.0))
        z = (lax.dot_general(z_mx, wl1_ref[0:H], (((1,), (0,)), ((), ())),
                             preferred_element_type=_f32)
             + lax.dot_general(z_mean, wl1_ref[H:2 * H],
                               (((1,), (0,)), ((), ())),
                               preferred_element_type=_f32)
             + bl1_ref[...])
        z = jnp.maximum(z, 0.0)
        out_ref[...] = lax.dot_general(z, wl2_ref[...],
                                       (((1,), (0,)), ((), ())),
                                       preferred_element_type=_f32) + bl2_ref[...]


def _dense2_call(aggp, x, batch_col, wrel, brel_row, wroot,
                 mx1, sum1, cnt, wl1, bl1_row, wl2, bl2_row):
    grid = N // RB
    return pl.pallas_call(
        _dense2_body,
        grid=(grid,),
        in_specs=[
            pl.BlockSpec((NC, RB, D), lambda i: (0, i, 0)),
            pl.BlockSpec((RB, D), lambda i: (i, 0)),
            pl.BlockSpec((RB, 1), lambda i: (i, 0)),
            pl.BlockSpec((D, H), lambda i: (0, 0)),
            pl.BlockSpec((1, H), lambda i: (0, 0)),
            pl.BlockSpec((D, H), lambda i: (0, 0)),
            pl.BlockSpec((B, H), lambda i: (0, 0)),
            pl.BlockSpec((B, H), lambda i: (0, 0)),
            pl.BlockSpec((B, H), lambda i: (0, 0)),
            pl.BlockSpec((2 * H, H), lambda i: (0, 0)),
            pl.BlockSpec((1, H), lambda i: (0, 0)),
            pl.BlockSpec((H, C), lambda i: (0, 0)),
            pl.BlockSpec((1, C), lambda i: (0, 0)),
        ],
        out_specs=pl.BlockSpec((B, C), lambda i: (0, 0)),
        out_shape=jax.ShapeDtypeStruct((B, C), _f32),
        scratch_shapes=[pltpu.VMEM((B, H), _f32), pltpu.VMEM((B, H), _f32)],
        compiler_params=pltpu.CompilerParams(
            dimension_semantics=("arbitrary",)),
    )(aggp, x, batch_col, wrel, brel_row, wroot, mx1, sum1, cnt,
      wl1, bl1_row, wl2, bl2_row)


def kernel(x, edge_index, edge_a, batch, W_be, b_be, W1_rel, b1_rel, W1_root,
           W2_rel, b2_rel, W2_root, Wl1, bl1, Wl2, bl2):
    src = edge_index[0]
    dst3 = edge_index[1].reshape(E // SUB, 1, SUB)

    eaT3 = edge_a.T.reshape(BOND, EW_ROWS, EW_COLS)
    ew = _ew_call(eaT3, W_be.reshape(1, BOND), b_be.reshape(1, 1)).reshape(E)

    zeros = jnp.zeros((RPS, D), _f32)
    batch_col = batch.reshape(N, 1)

    aggp1 = _sc_segment(x, src, dst3, ew, zeros)
    r1, sum1, max1, cnt = _dense_call(aggp1, x, batch_col, W1_rel,
                                      b1_rel.reshape(1, H), W1_root)

    aggp2 = _sc_segment(r1, src, dst3, ew, zeros)
    return _dense2_call(aggp2, r1, batch_col, W2_rel, b2_rel.reshape(1, H),
                        W2_root, max1, sum1, cnt, Wl1, bl1.reshape(1, H),
                        Wl2, bl2.reshape(1, C))


# speedup vs baseline: 1.0486x; 1.0069x over previous
import functools

import jax
import jax.numpy as jnp
from jax import lax
from jax.experimental import pallas as pl
from jax.experimental.pallas import tpu as pltpu
from jax.experimental.pallas import tpu_sc as plsc

N = 10000
E = 320000
D = 128
H = 128
C = 10
B = 64
BOND = 10

NC = 2
NS = 16
NW = NC * NS

EPW = E // NW
SUB = 80
WIN = 2000
WSUB = WIN // SUB
NSTG = EPW // WIN
RPS = 624
REM = N - NS * RPS

_f32 = jnp.float32
_i32 = jnp.int32


def _sc_body(vals_hbm, src_hbm, dst_hbm, ew_hbm, zeros_hbm, out_hbm,
             src_v, dst_v, ew_v, rows_v, scaled_v, agg_sh,
             gsem0, gsem1, ssem0, ssem1):
    c = lax.axis_index("c")
    s = lax.axis_index("s")
    wid = s * NC + c
    gsems = (gsem0, gsem1)
    ssems = (ssem0, ssem1)

    pltpu.sync_copy(zeros_hbm, agg_sh.at[pl.ds(s * RPS, RPS)])

    @pl.when(s == NS - 1)
    def _():
        pltpu.sync_copy(zeros_hbm.at[pl.ds(0, REM)],
                        agg_sh.at[pl.ds(NS * RPS, REM)])

    plsc.subcore_barrier()

    ebase = wid * EPW

    def g_copy(r, b):
        off = pl.multiple_of(r * SUB, SUB)
        return pltpu.make_async_copy(vals_hbm.at[src_v.at[pl.ds(off, SUB)]],
                                     rows_v.at[b], gsems[b])

    def s_copy(r, b):
        return pltpu.make_async_copy(scaled_v.at[b],
                                     agg_sh.at[dst_v.at[r, 0]], ssems[b])

    def scale(r, b):
        rbase = pl.multiple_of(r * SUB, SUB)

        @plsc.parallel_loop(0, SUB, step=1, unroll=8)
        def _(e):
            egrp = (e // 16) * 16
            ew16 = ew_v[pl.ds(pl.multiple_of(rbase + egrp, 16), 16)]
            lane = jnp.full((16,), e - egrp, _i32)
            splat = lax.gather(
                ew16, lane[:, None],
                lax.GatherDimensionNumbers(offset_dims=(),
                                           collapsed_slice_dims=(0,),
                                           start_index_map=(0,)),
                slice_sizes=(1,),
                mode=lax.GatherScatterMode.PROMISE_IN_BOUNDS)
            for cb in range(D // 32):
                w = plsc.bitcast(rows_v.at[b][e, pl.ds(32 * cb, 32)], _i32)
                va = plsc.bitcast(w << 16, _f32)
                vb = plsc.bitcast(w & jnp.int32(-65536), _f32)
                scaled_v.at[b][e, pl.ds(32 * cb, 16)] = va * splat
                scaled_v.at[b][e, pl.ds(32 * cb + 16, 16)] = vb * splat

    def s_start(r, b):
        pltpu.async_copy(scaled_v.at[b], agg_sh.at[dst_v.at[r, 0]], ssems[b],
                         add=True)

    for stage in range(NSTG):
        w0 = ebase + stage * WIN
        pltpu.sync_copy(src_hbm.at[pl.ds(w0, WIN)], src_v)
        pltpu.sync_copy(ew_hbm.at[pl.ds(w0, WIN)], ew_v)
        pltpu.sync_copy(dst_hbm.at[pl.ds(w0 // SUB, WSUB)], dst_v)

        b0 = stage % 2
        b1 = (stage + 1) % 2
        g_copy(0, b0).start()
        g_copy(1, b1).start()

        def pair_body(i, carry, _b0=b0, _b1=b1):
            for b, bb in ((0, _b0), (1, _b1)):
                r = 2 * i + b
                g_copy(r, bb).wait()

                @pl.when(r >= 2)
                def _():
                    s_copy(r - 2, bb).wait()

                scale(r, bb)
                s_start(r, bb)

                @pl.when(r + 2 <= WSUB - 1)
                def _():
                    g_copy(r + 2, bb).start()
            return carry

        lax.fori_loop(0, (WSUB - 1) // 2, pair_body, 0)

        rl = WSUB - 1
        g_copy(rl, b0).wait()
        s_copy(rl - 2, b0).wait()
        scale(rl, b0)
        s_start(rl, b0)
        s_copy(rl - 1, b1).wait()
        s_copy(rl, b0).wait()

    plsc.subcore_barrier()
    pltpu.sync_copy(agg_sh.at[pl.ds(s * RPS, RPS)],
                    out_hbm.at[c].at[pl.ds(s * RPS, RPS)])

    @pl.when(s == NS - 1)
    def _():
        pltpu.sync_copy(agg_sh.at[pl.ds(NS * RPS, REM)],
                        out_hbm.at[c].at[pl.ds(NS * RPS, REM)])


def _sc_segment(vals, src, dst3, ew, zeros):
    mesh = plsc.VectorSubcoreMesh(core_axis_name="c", subcore_axis_name="s",
                                  num_cores=NC, num_subcores=NS)
    fn = pl.kernel(
        _sc_body,
        out_type=jax.ShapeDtypeStruct((NC, N, D), _f32),
        mesh=mesh,
        scratch_types=[
            pltpu.VMEM((WIN,), _i32),
            pltpu.VMEM((WSUB, 1, SUB), _i32),
            pltpu.VMEM((WIN,), _f32),
            pltpu.VMEM((2, SUB, D), jnp.bfloat16),
            pltpu.VMEM((2, SUB, D), _f32),
            pltpu.VMEM_SHARED((N, D), _f32),
            pltpu.SemaphoreType.DMA,
            pltpu.SemaphoreType.DMA,
            pltpu.SemaphoreType.DMA,
            pltpu.SemaphoreType.DMA,
        ],
        compiler_params=pltpu.CompilerParams(needs_layout_passes=False,
                                             use_tc_tiling_on_sc=False),
    )
    return fn(vals, src, dst3, ew, zeros)


EW_ROWS = 640
EW_COLS = 500
EW_BLK_R = 80


def _ew_body(ea_ref, w_ref, b_ref, out_ref):
    acc = ea_ref[0] * w_ref[0:1, 0:1]
    for k in range(1, BOND):
        acc += ea_ref[k] * w_ref[0:1, k:k + 1]
    out_ref[...] = acc + b_ref[0:1, 0:1]


def _ew_call(eaT3, w_row, b11):
    grid = EW_ROWS // EW_BLK_R
    return pl.pallas_call(
        _ew_body,
        grid=(grid,),
        in_specs=[
            pl.BlockSpec((BOND, EW_BLK_R, EW_COLS), lambda i: (0, i, 0)),
            pl.BlockSpec((1, BOND), lambda i: (0, 0)),
            pl.BlockSpec((1, 1), lambda i: (0, 0)),
        ],
        out_specs=pl.BlockSpec((EW_BLK_R, EW_COLS), lambda i: (i, 0)),
        out_shape=jax.ShapeDtypeStruct((EW_ROWS, EW_COLS), _f32),
    )(eaT3, w_row, b11)


RB = 1000


def _dense_body(aggp_ref, x_ref, batch_ref, wrel_ref, brel_ref, wroot_ref,
                r_ref, sum_ref, max_ref, cnt_ref):
    i = pl.program_id(0)
    agg = aggp_ref[0] + aggp_ref[1]
    h = (lax.dot_general(agg, wrel_ref[...], (((1,), (0,)), ((), ())),
                         preferred_element_type=_f32)
         + brel_ref[...]
         + lax.dot_general(x_ref[...], wroot_ref[...], (((1,), (0,)), ((), ())),
                           preferred_element_type=_f32))
    r_ref[...] = jnp.maximum(h, 0.0)

    onehot = (batch_ref[...] ==
              lax.broadcasted_iota(_i32, (1, B), 1)).astype(_f32)
    sums = lax.dot_general(onehot, h, (((0,), (0,)), ((), ())),
                           preferred_element_type=_f32)
    cnts = lax.dot_general(onehot, jnp.ones((RB, D), _f32),
                           (((0,), (0,)), ((), ())),
                           preferred_element_type=_f32)

    @pl.when(i == 0)
    def _():
        sum_ref[...] = jnp.zeros_like(sum_ref)
        cnt_ref[...] = jnp.zeros_like(cnt_ref)
        max_ref[...] = jnp.full_like(max_ref, -jnp.inf)

    sum_ref[...] += sums
    cnt_ref[...] += cnts

    lo = batch_ref[0, 0]
    hi = batch_ref[RB - 1, 0]
    giota = lax.broadcasted_iota(_i32, (B, 1), 0)

    def gbody(g, carry):
        m = batch_ref[...] == g
        mg = jnp.max(jnp.where(m, h, -jnp.inf), axis=0, keepdims=True)
        max_ref[...] = jnp.where(giota == g,
                                 jnp.maximum(max_ref[...], mg), max_ref[...])
        return carry

    lax.fori_loop(lo, hi + 1, gbody, 0)


def _dense_call(aggp, x, batch_col, wrel, brel_row, wroot):
    grid = N // RB
    return pl.pallas_call(
        _dense_body,
        grid=(grid,),
        in_specs=[
            pl.BlockSpec((NC, RB, D), lambda i: (0, i, 0)),
            pl.BlockSpec((RB, D), lambda i: (i, 0)),
            pl.BlockSpec((RB, 1), lambda i: (i, 0)),
            pl.BlockSpec((D, H), lambda i: (0, 0)),
            pl.BlockSpec((1, H), lambda i: (0, 0)),
            pl.BlockSpec((D, H), lambda i: (0, 0)),
        ],
        out_specs=[
            pl.BlockSpec((RB, H), lambda i: (i, 0)),
            pl.BlockSpec((B, H), lambda i: (0, 0)),
            pl.BlockSpec((B, H), lambda i: (0, 0)),
            pl.BlockSpec((B, H), lambda i: (0, 0)),
        ],
        out_shape=[
            jax.ShapeDtypeStruct((N, H), _f32),
            jax.ShapeDtypeStruct((B, H), _f32),
            jax.ShapeDtypeStruct((B, H), _f32),
            jax.ShapeDtypeStruct((B, H), _f32),
        ],
        compiler_params=pltpu.CompilerParams(
            dimension_semantics=("arbitrary",)),
    )(aggp, x, batch_col, wrel, brel_row, wroot)


def _dense2_body(aggp_ref, x_ref, batch_ref, wrel_ref, brel_ref, wroot_ref,
                 mx1_ref, sum1_ref, cnt_ref, wl1_ref, bl1_ref, wl2_ref,
                 bl2_ref, out_ref, sum_s, max_s):
    i = pl.program_id(0)
    agg = aggp_ref[0] + aggp_ref[1]
    h = (lax.dot_general(agg, wrel_ref[...], (((1,), (0,)), ((), ())),
                         preferred_element_type=_f32)
         + brel_ref[...]
         + lax.dot_general(x_ref[...], wroot_ref[...], (((1,), (0,)), ((), ())),
                           preferred_element_type=_f32))

    onehot = (batch_ref[...] ==
              lax.broadcasted_iota(_i32, (1, B), 1)).astype(_f32)
    sums = lax.dot_general(onehot, h, (((0,), (0,)), ((), ())),
                           preferred_element_type=_f32)

    @pl.when(i == 0)
    def _():
        sum_s[...] = jnp.zeros_like(sum_s)
        max_s[...] = jnp.full_like(max_s, -jnp.inf)

    sum_s[...] += sums

    lo = batch_ref[0, 0]
    hi = batch_ref[RB - 1, 0]
    giota = lax.broadcasted_iota(_i32, (B, 1), 0)

    def gbody(g, carry):
        m = batch_ref[...] == g
        mg = jnp.max(jnp.where(m, h, -jnp.inf), axis=0, keepdims=True)
        max_s[...] = jnp.where(giota == g,
                               jnp.maximum(max_s[...], mg), max_s[...])
        return carry

    lax.fori_loop(lo, hi + 1, gbody, 0)

    @pl.when(i == N // RB - 1)
    def _():
        z_mx = mx1_ref[...] + max_s[...]
        z_mean = ((sum1_ref[...] + sum_s[...])
                  / jnp.maximum(cnt_ref[...], 1.0))
        z = (lax.dot_general(z_mx, wl1_ref[0:H], (((1,), (0,)), ((), ())),
                             preferred_element_type=_f32)
             + lax.dot_general(z_mean, wl1_ref[H:2 * H],
                               (((1,), (0,)), ((), ())),
                               preferred_element_type=_f32)
             + bl1_ref[...])
        z = jnp.maximum(z, 0.0)
        out_ref[...] = lax.dot_general(z, wl2_ref[...],
                                       (((1,), (0,)), ((), ())),
                                       preferred_element_type=_f32) + bl2_ref[...]


def _dense2_call(aggp, x, batch_col, wrel, brel_row, wroot,
                 mx1, sum1, cnt, wl1, bl1_row, wl2, bl2_row):
    grid = N // RB
    return pl.pallas_call(
        _dense2_body,
        grid=(grid,),
        in_specs=[
            pl.BlockSpec((NC, RB, D), lambda i: (0, i, 0)),
            pl.BlockSpec((RB, D), lambda i: (i, 0)),
            pl.BlockSpec((RB, 1), lambda i: (i, 0)),
            pl.BlockSpec((D, H), lambda i: (0, 0)),
            pl.BlockSpec((1, H), lambda i: (0, 0)),
            pl.BlockSpec((D, H), lambda i: (0, 0)),
            pl.BlockSpec((B, H), lambda i: (0, 0)),
            pl.BlockSpec((B, H), lambda i: (0, 0)),
            pl.BlockSpec((B, H), lambda i: (0, 0)),
            pl.BlockSpec((2 * H, H), lambda i: (0, 0)),
            pl.BlockSpec((1, H), lambda i: (0, 0)),
            pl.BlockSpec((H, C), lambda i: (0, 0)),
            pl.BlockSpec((1, C), lambda i: (0, 0)),
        ],
        out_specs=pl.BlockSpec((B, C), lambda i: (0, 0)),
        out_shape=jax.ShapeDtypeStruct((B, C), _f32),
        scratch_shapes=[pltpu.VMEM((B, H), _f32), pltpu.VMEM((B, H), _f32)],
        compiler_params=pltpu.CompilerParams(
            dimension_semantics=("arbitrary",)),
    )(aggp, x, batch_col, wrel, brel_row, wroot, mx1, sum1, cnt,
      wl1, bl1_row, wl2, bl2_row)


def kernel(x, edge_index, edge_a, batch, W_be, b_be, W1_rel, b1_rel, W1_root,
           W2_rel, b2_rel, W2_root, Wl1, bl1, Wl2, bl2):
    src = edge_index[0]
    dst3 = edge_index[1].reshape(E // SUB, 1, SUB)

    eaT3 = edge_a.T.reshape(BOND, EW_ROWS, EW_COLS)
    ew = _ew_call(eaT3, W_be.reshape(1, BOND), b_be.reshape(1, 1)).reshape(E)

    zeros = jnp.zeros((RPS, D), _f32)
    batch_col = batch.reshape(N, 1)

    perm = jnp.array([32 * cb + 2 * j + p
                      for cb in range(D // 32) for p in (0, 1)
                      for j in range(16)], dtype=_i32)
    w1rel_p = W1_rel[perm, :]
    w2rel_p = W2_rel[perm, :]

    aggp1 = _sc_segment(x.astype(jnp.bfloat16), src, dst3, ew, zeros)
    r1, sum1, max1, cnt = _dense_call(aggp1, x, batch_col, w1rel_p,
                                      b1_rel.reshape(1, H), W1_root)

    aggp2 = _sc_segment(r1.astype(jnp.bfloat16), src, dst3, ew, zeros)
    return _dense2_call(aggp2, r1, batch_col, w2rel_p, b2_rel.reshape(1, H),
                        W2_root, max1, sum1, cnt, Wl1, bl1.reshape(1, H),
                        Wl2, bl2.reshape(1, C))
